# baseline (speedup 1.0000x reference)
import functools

import jax
import jax.numpy as jnp
from jax import lax
from jax.experimental import pallas as pl
from jax.experimental.pallas import tpu as pltpu
from jax.experimental.pallas import tpu_sc as plsc

_V = 1000000
_D = 64
_R = 4096
_C = 200
_NW = 32
_NBUF = 2

_QB = 25
_NCH = _V // 8 // _QB
_A_ITERS = (_NCH + _NW - 1) // _NW

_BLK = 128
_NIB = _R // _BLK
_NUNITS = _C * _NIB
_B_ITERS = _NUNITS // _NW


def _make_widen():
    mesh = plsc.VectorSubcoreMesh(core_axis_name="c", subcore_axis_name="s")

    @functools.partial(
        pl.kernel,
        mesh=mesh,
        out_type=jax.ShapeDtypeStruct((_V // 8, 8, 128), jnp.float32),
        scratch_types=[
            pltpu.VMEM((_NBUF, _QB, 8, _D), jnp.float32),
            pltpu.VMEM((_NBUF, _QB, 8, 128), jnp.float32),
        ] + [pltpu.SemaphoreType.DMA] * (2 * _NBUF),
        compiler_params=pltpu.CompilerParams(
            disable_bounds_checks=True,
            disable_semaphore_checks=True,
        ),
    )
    def widen(tbl3, out, stage_a, stage_b, *sems):
        sem_i = sems[:_NBUF]
        sem_o = sems[_NBUF:]
        wid = lax.axis_index("s") * 2 + lax.axis_index("c")

        def i_copy(ch, b):
            return pltpu.make_async_copy(
                tbl3.at[pl.ds(ch * _QB, _QB)], stage_a.at[b], sem_i[b])

        def o_copy(ch, b):
            return pltpu.make_async_copy(
                stage_b.at[b], out.at[pl.ds(ch * _QB, _QB)], sem_o[b])

        def repack(b):
            def row(q, carry):
                for s in range(8):
                    for g in range(_D // 16):
                        stage_b[b, q, s, pl.ds(g * 16, 16)] = (
                            stage_a[b, q, s, pl.ds(g * 16, 16)])
                return carry
            lax.fori_loop(0, _QB, row, 0)

        for b in range(_NBUF):
            ch = wid + _NW * b
            @pl.when(ch < _NCH)
            def _():
                i_copy(ch, b).start()

        def group(k, carry):
            for b in range(_NBUF):
                ch = wid + _NW * (k * _NBUF + b)
                nch = ch + _NW * _NBUF

                @pl.when(ch < _NCH)
                def _():
                    i_copy(ch, b).wait()
                    repack(b)
                    o_copy(ch, b).start()
                    o_copy(ch, b).wait()

                @pl.when(nch < _NCH)
                def _():
                    i_copy(nch, b).start()
            return carry

        lax.fori_loop(0, (_A_ITERS + _NBUF - 1) // _NBUF, group, 0)

    return widen


def _make_gather():
    mesh = plsc.VectorSubcoreMesh(core_axis_name="c", subcore_axis_name="s")

    @functools.partial(
        pl.kernel,
        mesh=mesh,
        out_type=jax.ShapeDtypeStruct((_R, _C, _D), jnp.float32),
        scratch_types=[
            pltpu.VMEM((_NBUF, _BLK), jnp.int32),
            pltpu.VMEM((_NBUF, _BLK, 128), jnp.float32),
            pltpu.VMEM((_NBUF, _BLK, _D), jnp.float32),
        ] + [pltpu.SemaphoreType.DMA] * (3 * _NBUF),
        compiler_params=pltpu.CompilerParams(
            disable_bounds_checks=True,
            disable_semaphore_checks=True,
        ),
    )
    def gather(x_t, tableP, out, idx_v, rows_w, rows_n, *sems):
        sem_x = sems[:_NBUF]
        sem_g = sems[_NBUF:2 * _NBUF]
        sem_o = sems[2 * _NBUF:]
        wid = lax.axis_index("s") * 2 + lax.axis_index("c")

        def unit(u):
            return u // _NIB, (u % _NIB) * _BLK

        def x_copy(u, b):
            j, i0 = unit(u)
            return pltpu.make_async_copy(
                x_t.at[j, pl.ds(i0, _BLK)], idx_v.at[b], sem_x[b])

        def g_copy(b):
            return pltpu.make_async_copy(
                tableP.at[idx_v.at[b]], rows_w.at[b], sem_g[b])

        def o_copy(u, b):
            j, i0 = unit(u)
            return pltpu.make_async_copy(
                rows_n.at[b], out.at[pl.ds(i0, _BLK), j], sem_o[b])

        def narrow(b):
            def row(r, carry):
                for g in range(_D // 16):
                    rows_n[b, r, pl.ds(g * 16, 16)] = (
                        rows_w[b, r, pl.ds(g * 16, 16)])
                return carry
            lax.fori_loop(0, _BLK, row, 0)

        for b in range(_NBUF):
            x_copy(wid + _NW * b, b).start()

        def group(k, carry):
            for b in range(_NBUF):
                u = wid + _NW * (k * _NBUF + b)
                nu = u + _NW * _NBUF
                x_copy(u, b).wait()
                g_copy(b).start()
                g_copy(b).wait()

                @pl.when(nu < _NUNITS)
                def _():
                    x_copy(nu, b).start()

                narrow(b)
                o_copy(u, b).start()
                o_copy(u, b).wait()
            return carry

        lax.fori_loop(0, _B_ITERS // _NBUF, group, 0)

    return gather


_widen = _make_widen()
_gather = _make_gather()


def kernel(x, table):
    tbl3 = table.reshape(_V // 8, 8, _D)
    tableP = _widen(tbl3).reshape(_V, 128)
    x_t = jnp.transpose(x)
    return _gather(x_t, tableP)
